# Initial kernel scaffold; baseline (speedup 1.0000x reference)
import jax
import jax.numpy as jnp
from jax.experimental import pallas as pl


def kernel(x, edge_index, Wl1, Wr1, att1, b1, Wl2, Wr2, att2, b2, Wl3, Wr3, att3, b3, Wl4, Wr4, att4, b4, Wl5, Wr5, att5, b5, R1, rb1, R2, rb2, R3, rb3, R4, rb4):
    raise NotImplementedError("write your pallas kernel here")


# speedup vs baseline: 14.6513x; 14.6513x over previous
import functools

import jax
import jax.numpy as jnp
from jax import lax
from jax.experimental import pallas as pl
from jax.experimental.pallas import tpu as pltpu

_N = 10000
_E = 320000
_H = 8
_HID = 8
_NC = 40
_BLK = 2000


def _leaky(v):
    return jnp.where(v >= 0, v, 0.2 * v)


def _dot(a, b):
    return jnp.dot(a, b, preferred_element_type=jnp.float32)


def _lin_body(h_ref, wl_ref, wr_ref, xl_ref, xr_ref):
    h = h_ref[...]
    xl_ref[...] = _dot(h, wl_ref[...])
    xr_ref[...] = _dot(h, wr_ref[...])


def _lin(h, Wl, Wr):
    n, din = h.shape
    w = Wl.shape[1]
    return pl.pallas_call(
        _lin_body,
        grid=(n // _BLK,),
        in_specs=[
            pl.BlockSpec((_BLK, din), lambda i: (i, 0)),
            pl.BlockSpec((din, w), lambda i: (0, 0)),
            pl.BlockSpec((din, w), lambda i: (0, 0)),
        ],
        out_specs=[
            pl.BlockSpec((_BLK, w), lambda i: (i, 0)),
            pl.BlockSpec((_BLK, w), lambda i: (i, 0)),
        ],
        out_shape=[jax.ShapeDtypeStruct((n, w), jnp.float32)] * 2,
    )(h, Wl, Wr)


def _mid_body(acc_ref, xl_ref, xr_ref, hprev_ref, a_ref, k_ref, b_ref,
              r_ref, rb_ref, h_ref):
    xl = xl_ref[...]
    xr = xr_ref[...]
    w = xl.shape[1]
    exs = jnp.exp(_dot(_leaky(xl + xr), a_ref[...]))
    acc = acc_ref[0] + acc_ref[1]
    num = acc[:, :w] + _dot(exs, k_ref[...]) * xl
    den = acc[:, w:] + exs
    denx = _dot(den, k_ref[...]) + 1e-16
    gat = num / denx + b_ref[...]
    h_ref[...] = _leaky(gat + _dot(hprev_ref[...], r_ref[...]) + rb_ref[...])


def _mid(acc, xl, xr, hprev, A, K, b, R, rb):
    n, w = xl.shape
    din = hprev.shape[1]
    f = w + _H
    return pl.pallas_call(
        _mid_body,
        grid=(n // _BLK,),
        in_specs=[
            pl.BlockSpec((2, _BLK, f), lambda i: (0, i, 0)),
            pl.BlockSpec((_BLK, w), lambda i: (i, 0)),
            pl.BlockSpec((_BLK, w), lambda i: (i, 0)),
            pl.BlockSpec((_BLK, din), lambda i: (i, 0)),
            pl.BlockSpec((w, _H), lambda i: (0, 0)),
            pl.BlockSpec((_H, w), lambda i: (0, 0)),
            pl.BlockSpec((1, w), lambda i: (0, 0)),
            pl.BlockSpec((din, w), lambda i: (0, 0)),
            pl.BlockSpec((1, w), lambda i: (0, 0)),
        ],
        out_specs=pl.BlockSpec((_BLK, w), lambda i: (i, 0)),
        out_shape=jax.ShapeDtypeStruct((n, w), jnp.float32),
    )(acc, xl, xr, hprev, A, K, b, R, rb)


def _final_body(acca_ref, accb_ref, xla_ref, xra_ref, xlb_ref, xrb_ref,
                aa_ref, ab_ref, k_ref, s_ref, b_ref, out_ref):
    w = xla_ref.shape[1]

    def gat_chunk(acc_ref, xl_ref, xr_ref, a_ref):
        xl = xl_ref[...]
        exs = jnp.exp(_dot(_leaky(xl + xr_ref[...]), a_ref[...]))
        acc = acc_ref[0] + acc_ref[1]
        num = acc[:, :w] + _dot(exs, k_ref[...]) * xl
        den = acc[:, w:] + exs
        return num / (_dot(den, k_ref[...]) + 1e-16)

    ga = gat_chunk(acca_ref, xla_ref, xra_ref, aa_ref)
    gb = gat_chunk(accb_ref, xlb_ref, xrb_ref, ab_ref)
    out = (_dot(ga, s_ref[...]) + _dot(gb, s_ref[...])) * 0.125 + b_ref[...]
    mx = jnp.max(out, axis=1, keepdims=True)
    lse = jnp.log(jnp.sum(jnp.exp(out - mx), axis=1, keepdims=True)) + mx
    out_ref[...] = out - lse


def _final(acca, accb, xla, xra, xlb, xrb, Aa, Ab, K, S, b5):
    n, w = xla.shape
    hh = _H // 2
    f = w + hh
    return pl.pallas_call(
        _final_body,
        grid=(n // _BLK,),
        in_specs=[
            pl.BlockSpec((2, _BLK, f), lambda i: (0, i, 0)),
            pl.BlockSpec((2, _BLK, f), lambda i: (0, i, 0)),
            pl.BlockSpec((_BLK, w), lambda i: (i, 0)),
            pl.BlockSpec((_BLK, w), lambda i: (i, 0)),
            pl.BlockSpec((_BLK, w), lambda i: (i, 0)),
            pl.BlockSpec((_BLK, w), lambda i: (i, 0)),
            pl.BlockSpec((w, hh), lambda i: (0, 0)),
            pl.BlockSpec((w, hh), lambda i: (0, 0)),
            pl.BlockSpec((hh, w), lambda i: (0, 0)),
            pl.BlockSpec((w, _NC), lambda i: (0, 0)),
            pl.BlockSpec((1, _NC), lambda i: (0, 0)),
        ],
        out_specs=pl.BlockSpec((_BLK, _NC), lambda i: (i, 0)),
        out_shape=jax.ShapeDtypeStruct((n, _NC), jnp.float32),
    )(acca, accb, xla, xra, xlb, xrb, Aa, Ab, K, S, b5)


def _edge_jnp(xl, xr, A, K, src, dst):
    m = _leaky(xl[src] + xr[dst])
    exs = jnp.exp(_dot(m, A))
    vals = jnp.concatenate([_dot(exs, K) * xl[src], exs], axis=1)
    acc = jax.ops.segment_sum(vals, dst, num_segments=_N)
    return jnp.stack([acc, jnp.zeros_like(acc)])


def _att_mats(att):
    h, ch = att.shape
    A = (att[:, :, None] * jnp.eye(h, dtype=jnp.float32)[:, None, :]).reshape(
        h * ch, h)
    K = jnp.kron(jnp.eye(h, dtype=jnp.float32),
                 jnp.ones((1, ch), jnp.float32))
    return A, K


def kernel(x, edge_index, Wl1, Wr1, att1, b1, Wl2, Wr2, att2, b2, Wl3, Wr3,
           att3, b3, Wl4, Wr4, att4, b4, Wl5, Wr5, att5, b5, R1, rb1, R2,
           rb2, R3, rb3, R4, rb4):
    src = edge_index[0]
    dst = edge_index[1]

    A1, K1 = _att_mats(att1)
    A2, K2 = _att_mats(att2)
    A3, K3 = _att_mats(att3)
    A4, K4 = _att_mats(att4)
    A5a, K5 = _att_mats(att5[: _H // 2])
    A5b, _ = _att_mats(att5[_H // 2:])
    S5 = jnp.tile(jnp.eye(_NC, dtype=jnp.float32), (_H // 2, 1))

    xl, xr = _lin(x, Wl1, Wr1)
    acc = _edge_jnp(xl, xr, A1, K1, src, dst)
    h = _mid(acc, xl, xr, x, A1, K1, b1.reshape(1, -1), R1, rb1.reshape(1, -1))

    for (Wl, Wr, A, K, b, R, rb) in (
            (Wl2, Wr2, A2, K2, b2, R2, rb2),
            (Wl3, Wr3, A3, K3, b3, R3, rb3),
            (Wl4, Wr4, A4, K4, b4, R4, rb4)):
        xl, xr = _lin(h, Wl, Wr)
        acc = _edge_jnp(xl, xr, A, K, src, dst)
        h = _mid(acc, xl, xr, h, A, K, b.reshape(1, -1), R, rb.reshape(1, -1))

    hw = _H // 2 * _NC
    xla, xra = _lin(h, Wl5[:, :hw], Wr5[:, :hw])
    xlb, xrb = _lin(h, Wl5[:, hw:], Wr5[:, hw:])
    acca = _edge_jnp(xla, xra, A5a, K5, src, dst)
    accb = _edge_jnp(xlb, xrb, A5b, K5, src, dst)
    return _final(acca, accb, xla, xra, xlb, xrb, A5a, A5b, K5, S5,
                  b5.reshape(1, -1))


# speedup vs baseline: 31.4628x; 2.1474x over previous
import functools

import jax
import jax.numpy as jnp
from jax import lax
from jax.experimental import pallas as pl
from jax.experimental.pallas import tpu as pltpu
from jax.experimental.pallas import tpu_sc as plsc

_N = 10000
_E = 320000
_H = 8
_HID = 8
_NC = 40
_BLK = 2000

_NCORE = 2
_NSUB = 16
_NW = _NCORE * _NSUB
_EW = _E // _NW
_C = 80
_CPW = _EW // _C
_KG = 5
_NGRP = _CPW // _KG


def _leaky(v):
    return jnp.where(v >= 0, v, 0.2 * v)


def _dot(a, b):
    return jnp.dot(a, b, preferred_element_type=jnp.float32)


def _lin_body(h_ref, wl_ref, wr_ref, xl_ref, xr_ref):
    h = h_ref[...]
    xl_ref[...] = _dot(h, wl_ref[...])
    xr_ref[...] = _dot(h, wr_ref[...])


def _lin(h, Wl, Wr):
    n, din = h.shape
    w = Wl.shape[1]
    return pl.pallas_call(
        _lin_body,
        grid=(n // _BLK,),
        in_specs=[
            pl.BlockSpec((_BLK, din), lambda i: (i, 0)),
            pl.BlockSpec((din, w), lambda i: (0, 0)),
            pl.BlockSpec((din, w), lambda i: (0, 0)),
        ],
        out_specs=[
            pl.BlockSpec((_BLK, w), lambda i: (i, 0)),
            pl.BlockSpec((_BLK, w), lambda i: (i, 0)),
        ],
        out_shape=[jax.ShapeDtypeStruct((n, w), jnp.float32)] * 2,
    )(h, Wl, Wr)


def _mid_body(acc_ref, xl_ref, xr_ref, hprev_ref, a_ref, k_ref, b_ref,
              r_ref, rb_ref, h_ref):
    xl = xl_ref[...]
    xr = xr_ref[...]
    w = xl.shape[1]
    exs = jnp.exp(_dot(_leaky(xl + xr), a_ref[...]))
    acc = acc_ref[0] + acc_ref[1]
    num = acc[:, :w] + _dot(exs, k_ref[...]) * xl
    den = acc[:, w:] + exs
    denx = _dot(den, k_ref[...]) + 1e-16
    gat = num / denx + b_ref[...]
    h_ref[...] = _leaky(gat + _dot(hprev_ref[...], r_ref[...]) + rb_ref[...])


def _mid(acc, xl, xr, hprev, A, K, b, R, rb):
    n, w = xl.shape
    din = hprev.shape[1]
    f = w + _H
    return pl.pallas_call(
        _mid_body,
        grid=(n // _BLK,),
        in_specs=[
            pl.BlockSpec((2, _BLK, f), lambda i: (0, i, 0)),
            pl.BlockSpec((_BLK, w), lambda i: (i, 0)),
            pl.BlockSpec((_BLK, w), lambda i: (i, 0)),
            pl.BlockSpec((_BLK, din), lambda i: (i, 0)),
            pl.BlockSpec((w, _H), lambda i: (0, 0)),
            pl.BlockSpec((_H, w), lambda i: (0, 0)),
            pl.BlockSpec((1, w), lambda i: (0, 0)),
            pl.BlockSpec((din, w), lambda i: (0, 0)),
            pl.BlockSpec((1, w), lambda i: (0, 0)),
        ],
        out_specs=pl.BlockSpec((_BLK, w), lambda i: (i, 0)),
        out_shape=jax.ShapeDtypeStruct((n, w), jnp.float32),
    )(acc, xl, xr, hprev, A, K, b, R, rb)


def _final_body(acca_ref, accb_ref, xla_ref, xra_ref, xlb_ref, xrb_ref,
                aa_ref, ab_ref, k_ref, s_ref, b_ref, out_ref):
    w = xla_ref.shape[1]

    def gat_chunk(acc_ref, xl_ref, xr_ref, a_ref):
        xl = xl_ref[...]
        exs = jnp.exp(_dot(_leaky(xl + xr_ref[...]), a_ref[...]))
        acc = acc_ref[0] + acc_ref[1]
        num = acc[:, :w] + _dot(exs, k_ref[...]) * xl
        den = acc[:, w:w + 4] + exs
        return num / (_dot(den, k_ref[...]) + 1e-16)

    ga = gat_chunk(acca_ref, xla_ref, xra_ref, aa_ref)
    gb = gat_chunk(accb_ref, xlb_ref, xrb_ref, ab_ref)
    out = (_dot(ga, s_ref[...]) + _dot(gb, s_ref[...])) * 0.125 + b_ref[...]
    mx = jnp.max(out, axis=1, keepdims=True)
    lse = jnp.log(jnp.sum(jnp.exp(out - mx), axis=1, keepdims=True)) + mx
    out_ref[...] = out - lse


def _final(acca, accb, xla, xra, xlb, xrb, Aa, Ab, K, S, b5):
    n, w = xla.shape
    hh = _H // 2
    f = w + hh + 4
    return pl.pallas_call(
        _final_body,
        grid=(n // _BLK,),
        in_specs=[
            pl.BlockSpec((2, _BLK, f), lambda i: (0, i, 0)),
            pl.BlockSpec((2, _BLK, f), lambda i: (0, i, 0)),
            pl.BlockSpec((_BLK, w), lambda i: (i, 0)),
            pl.BlockSpec((_BLK, w), lambda i: (i, 0)),
            pl.BlockSpec((_BLK, w), lambda i: (i, 0)),
            pl.BlockSpec((_BLK, w), lambda i: (i, 0)),
            pl.BlockSpec((w, hh), lambda i: (0, 0)),
            pl.BlockSpec((w, hh), lambda i: (0, 0)),
            pl.BlockSpec((hh, w), lambda i: (0, 0)),
            pl.BlockSpec((w, _NC), lambda i: (0, 0)),
            pl.BlockSpec((1, _NC), lambda i: (0, 0)),
        ],
        out_specs=pl.BlockSpec((_BLK, _NC), lambda i: (i, 0)),
        out_shape=jax.ShapeDtypeStruct((n, _NC), jnp.float32),
    )(acca, accb, xla, xra, xlb, xrb, Aa, Ab, K, S, b5)


def _gather_body(xl_hbm, xr_hbm, src_hbm, dst_hbm, outl, outr, idx_v, *rest):
    bufs = rest[:_KG]
    gsem, wsem = rest[_KG], rest[_KG + 1]
    wid = lax.axis_index("c") * _NSUB + lax.axis_index("s")
    for table, ind, out in ((xl_hbm, src_hbm, outl), (xr_hbm, dst_hbm, outr)):
        pltpu.sync_copy(ind.at[wid], idx_v)

        def grp(g, _):
            base = g * _KG
            for u in range(_KG):
                pltpu.async_copy(table.at[idx_v.at[base + u]], bufs[u], gsem)
            for u in range(_KG):
                pltpu.make_async_copy(
                    table.at[idx_v.at[base + u]], bufs[u], gsem).wait()
            row0 = (wid * _CPW + base) * _C
            for u in range(_KG):
                pltpu.async_copy(
                    bufs[u], out.at[pl.ds(row0 + u * _C, _C)], wsem)
            for u in range(_KG):
                pltpu.make_async_copy(
                    bufs[u], out.at[pl.ds(row0 + u * _C, _C)], wsem).wait()
            return 0

        lax.fori_loop(0, _NGRP, grp, 0)


def _gather_sc(xl, xr, src3, dst3):
    w = xl.shape[1]
    mesh = plsc.VectorSubcoreMesh(core_axis_name="c", subcore_axis_name="s")
    return pl.kernel(
        _gather_body,
        out_type=[jax.ShapeDtypeStruct((_E, w), jnp.float32)] * 2,
        mesh=mesh,
        compiler_params=pltpu.CompilerParams(use_tc_tiling_on_sc=False),
        scratch_types=(
            [pltpu.VMEM((_CPW, _C), jnp.int32)]
            + [pltpu.VMEM((_C, w), jnp.float32) for _ in range(_KG)]
            + [pltpu.SemaphoreType.DMA, pltpu.SemaphoreType.DMA]),
    )(xl, xr, src3, dst3)


def _scatter_body(vals_hbm, dst_hbm, z_hbm, out_hbm, idx_v, vb, acc_sp):
    cid = lax.axis_index("c")
    sid = lax.axis_index("s")
    wid = cid * _NSUB + sid

    pltpu.sync_copy(z_hbm, vb)

    def zc(i, _):
        ch = sid + i * _NSUB

        @pl.when(ch < _CPW)
        def _():
            pltpu.sync_copy(vb, acc_sp.at[pl.ds(ch * _C, _C)])
        return 0

    lax.fori_loop(0, (_CPW + _NSUB - 1) // _NSUB, zc, 0)
    plsc.subcore_barrier()

    pltpu.sync_copy(dst_hbm.at[wid], idx_v)

    def ch(j, _):
        row0 = (wid * _CPW + j) * _C
        pltpu.sync_copy(vals_hbm.at[pl.ds(row0, _C)], vb)
        pltpu.sync_copy(vb, acc_sp.at[idx_v.at[j]], add=True)
        return 0

    lax.fori_loop(0, _CPW, ch, 0)
    plsc.subcore_barrier()

    def dc(i, _):
        chn = sid + i * _NSUB

        @pl.when(chn < _CPW)
        def _():
            pltpu.sync_copy(acc_sp.at[pl.ds(chn * _C, _C)], vb)
            pltpu.sync_copy(
                vb, out_hbm.at[pl.ds(cid * _N + chn * _C, _C)])
        return 0

    lax.fori_loop(0, (_CPW + _NSUB - 1) // _NSUB, dc, 0)


def _scatter_sc(vals, dst3, zrows):
    f = vals.shape[1]
    mesh = plsc.VectorSubcoreMesh(core_axis_name="c", subcore_axis_name="s")
    out = pl.kernel(
        _scatter_body,
        out_type=jax.ShapeDtypeStruct((_NCORE * _N, f), jnp.float32),
        mesh=mesh,
        compiler_params=pltpu.CompilerParams(use_tc_tiling_on_sc=False),
        scratch_types=(
            pltpu.VMEM((_CPW, _C), jnp.int32),
            pltpu.VMEM((_C, f), jnp.float32),
            pltpu.VMEM_SHARED((_N, f), jnp.float32),
        ),
    )(vals, dst3, zrows)
    return out.reshape(_NCORE, _N, f)


def _emath_body(pad, xls_ref, xrs_ref, a_ref, k_ref, vals_ref):
    xls = xls_ref[...]
    exs = jnp.exp(_dot(_leaky(xls + xrs_ref[...]), a_ref[...]))
    parts = [_dot(exs, k_ref[...]) * xls, exs]
    if pad:
        parts.append(jnp.zeros((xls.shape[0], pad), jnp.float32))
    vals_ref[...] = jnp.concatenate(parts, axis=1)


def _emath(xls, xrs, A, K, pad):
    w = xls.shape[1]
    h = A.shape[1] + 0
    blk = 4000
    return pl.pallas_call(
        functools.partial(_emath_body, pad),
        grid=(_E // blk,),
        in_specs=[
            pl.BlockSpec((blk, w), lambda i: (i, 0)),
            pl.BlockSpec((blk, w), lambda i: (i, 0)),
            pl.BlockSpec((w, h), lambda i: (0, 0)),
            pl.BlockSpec((h, w), lambda i: (0, 0)),
        ],
        out_specs=pl.BlockSpec((blk, w + h + pad), lambda i: (i, 0)),
        out_shape=jax.ShapeDtypeStruct((_E, w + h + pad), jnp.float32),
    )(xls, xrs, A, K)


def _edge_sc(xl, xr, A, K, src3, dst3, zrows, pad=0):
    xls, xrs = _gather_sc(xl, xr, src3, dst3)
    vals = _emath(xls, xrs, A, K, pad)
    return _scatter_sc(vals, dst3, zrows)


def _att_mats(att):
    h, ch = att.shape
    A = (att[:, :, None] * jnp.eye(h, dtype=jnp.float32)[:, None, :]).reshape(
        h * ch, h)
    K = jnp.kron(jnp.eye(h, dtype=jnp.float32),
                 jnp.ones((1, ch), jnp.float32))
    return A, K


def kernel(x, edge_index, Wl1, Wr1, att1, b1, Wl2, Wr2, att2, b2, Wl3, Wr3,
           att3, b3, Wl4, Wr4, att4, b4, Wl5, Wr5, att5, b5, R1, rb1, R2,
           rb2, R3, rb3, R4, rb4):
    src3 = edge_index[0].reshape(_NW, _CPW, _C)
    dst3 = edge_index[1].reshape(_NW, _CPW, _C)
    z72 = jnp.zeros((_C, 64 + _H), jnp.float32)
    z168 = jnp.zeros((_C, 168), jnp.float32)

    A1, K1 = _att_mats(att1)
    A2, K2 = _att_mats(att2)
    A3, K3 = _att_mats(att3)
    A4, K4 = _att_mats(att4)
    A5a, K5 = _att_mats(att5[: _H // 2])
    A5b, _ = _att_mats(att5[_H // 2:])
    S5 = jnp.tile(jnp.eye(_NC, dtype=jnp.float32), (_H // 2, 1))

    xl, xr = _lin(x, Wl1, Wr1)
    acc = _edge_sc(xl, xr, A1, K1, src3, dst3, z72)
    h = _mid(acc, xl, xr, x, A1, K1, b1.reshape(1, -1), R1, rb1.reshape(1, -1))

    for (Wl, Wr, A, K, b, R, rb) in (
            (Wl2, Wr2, A2, K2, b2, R2, rb2),
            (Wl3, Wr3, A3, K3, b3, R3, rb3),
            (Wl4, Wr4, A4, K4, b4, R4, rb4)):
        xl, xr = _lin(h, Wl, Wr)
        acc = _edge_sc(xl, xr, A, K, src3, dst3, z72)
        h = _mid(acc, xl, xr, h, A, K, b.reshape(1, -1), R, rb.reshape(1, -1))

    hw = _H // 2 * _NC
    xla, xra = _lin(h, Wl5[:, :hw], Wr5[:, :hw])
    xlb, xrb = _lin(h, Wl5[:, hw:], Wr5[:, hw:])
    acca = _edge_sc(xla, xra, A5a, K5, src3, dst3, z168, pad=4)
    accb = _edge_sc(xlb, xrb, A5b, K5, src3, dst3, z168, pad=4)
    return _final(acca, accb, xla, xra, xlb, xrb, A5a, A5b, K5, S5,
                  b5.reshape(1, -1))


# speedup vs baseline: 33.7021x; 1.0712x over previous
import functools

import jax
import jax.numpy as jnp
from jax import lax
from jax.experimental import pallas as pl
from jax.experimental.pallas import tpu as pltpu
from jax.experimental.pallas import tpu_sc as plsc

_N = 10000
_E = 320000
_H = 8
_HID = 8
_NC = 40
_BLK = 2000

_NCORE = 2
_NSUB = 16
_NW = _NCORE * _NSUB
_EW = _E // _NW
_C = 40
_CPW = _EW // _C
_NBUF = 5
_UNROLL = 10


def _leaky(v):
    return jnp.where(v >= 0, v, 0.2 * v)


def _dot(a, b):
    return jnp.dot(a, b, preferred_element_type=jnp.float32)


def _lin_body(h_ref, wl_ref, wr_ref, xl_ref, xr_ref):
    h = h_ref[...]
    xl_ref[...] = _dot(h, wl_ref[...])
    xr_ref[...] = _dot(h, wr_ref[...])


def _lin(h, Wl, Wr):
    n, din = h.shape
    w = Wl.shape[1]
    return pl.pallas_call(
        _lin_body,
        grid=(n // _BLK,),
        in_specs=[
            pl.BlockSpec((_BLK, din), lambda i: (i, 0)),
            pl.BlockSpec((din, w), lambda i: (0, 0)),
            pl.BlockSpec((din, w), lambda i: (0, 0)),
        ],
        out_specs=[
            pl.BlockSpec((_BLK, w), lambda i: (i, 0)),
            pl.BlockSpec((_BLK, w), lambda i: (i, 0)),
        ],
        out_shape=[jax.ShapeDtypeStruct((n, w), jnp.float32)] * 2,
    )(h, Wl, Wr)


def _mid_body(acc_ref, xl_ref, xr_ref, hprev_ref, a_ref, k_ref, b_ref,
              r_ref, rb_ref, h_ref):
    xl = xl_ref[...]
    xr = xr_ref[...]
    w = xl.shape[1]
    exs = jnp.exp(_dot(_leaky(xl + xr), a_ref[...]))
    acc = acc_ref[0] + acc_ref[1]
    num = acc[:, :w] + _dot(exs, k_ref[...]) * xl
    den = acc[:, w:] + exs
    denx = _dot(den, k_ref[...]) + 1e-16
    gat = num / denx + b_ref[...]
    h_ref[...] = _leaky(gat + _dot(hprev_ref[...], r_ref[...]) + rb_ref[...])


def _mid(acc, xl, xr, hprev, A, K, b, R, rb):
    n, w = xl.shape
    din = hprev.shape[1]
    f = w + _H
    return pl.pallas_call(
        _mid_body,
        grid=(n // _BLK,),
        in_specs=[
            pl.BlockSpec((2, _BLK, f), lambda i: (0, i, 0)),
            pl.BlockSpec((_BLK, w), lambda i: (i, 0)),
            pl.BlockSpec((_BLK, w), lambda i: (i, 0)),
            pl.BlockSpec((_BLK, din), lambda i: (i, 0)),
            pl.BlockSpec((w, _H), lambda i: (0, 0)),
            pl.BlockSpec((_H, w), lambda i: (0, 0)),
            pl.BlockSpec((1, w), lambda i: (0, 0)),
            pl.BlockSpec((din, w), lambda i: (0, 0)),
            pl.BlockSpec((1, w), lambda i: (0, 0)),
        ],
        out_specs=pl.BlockSpec((_BLK, w), lambda i: (i, 0)),
        out_shape=jax.ShapeDtypeStruct((n, w), jnp.float32),
    )(acc, xl, xr, hprev, A, K, b, R, rb)


def _final_body(acca_ref, accb_ref, xla_ref, xra_ref, xlb_ref, xrb_ref,
                aa_ref, ab_ref, k_ref, s_ref, b_ref, out_ref):
    w = xla_ref.shape[1]

    def gat_chunk(acc_ref, xl_ref, xr_ref, a_ref):
        xl = xl_ref[...]
        exs = jnp.exp(_dot(_leaky(xl + xr_ref[...]), a_ref[...]))
        acc = acc_ref[0] + acc_ref[1]
        num = acc[:, :w] + _dot(exs, k_ref[...]) * xl
        den = acc[:, w:w + 4] + exs
        return num / (_dot(den, k_ref[...]) + 1e-16)

    ga = gat_chunk(acca_ref, xla_ref, xra_ref, aa_ref)
    gb = gat_chunk(accb_ref, xlb_ref, xrb_ref, ab_ref)
    out = (_dot(ga, s_ref[...]) + _dot(gb, s_ref[...])) * 0.125 + b_ref[...]
    mx = jnp.max(out, axis=1, keepdims=True)
    lse = jnp.log(jnp.sum(jnp.exp(out - mx), axis=1, keepdims=True)) + mx
    out_ref[...] = out - lse


def _final(acca, accb, xla, xra, xlb, xrb, Aa, Ab, K, S, b5):
    n, w = xla.shape
    hh = _H // 2
    f = w + hh + 4
    return pl.pallas_call(
        _final_body,
        grid=(n // _BLK,),
        in_specs=[
            pl.BlockSpec((2, _BLK, f), lambda i: (0, i, 0)),
            pl.BlockSpec((2, _BLK, f), lambda i: (0, i, 0)),
            pl.BlockSpec((_BLK, w), lambda i: (i, 0)),
            pl.BlockSpec((_BLK, w), lambda i: (i, 0)),
            pl.BlockSpec((_BLK, w), lambda i: (i, 0)),
            pl.BlockSpec((_BLK, w), lambda i: (i, 0)),
            pl.BlockSpec((w, hh), lambda i: (0, 0)),
            pl.BlockSpec((w, hh), lambda i: (0, 0)),
            pl.BlockSpec((hh, w), lambda i: (0, 0)),
            pl.BlockSpec((w, _NC), lambda i: (0, 0)),
            pl.BlockSpec((1, _NC), lambda i: (0, 0)),
        ],
        out_specs=pl.BlockSpec((_BLK, _NC), lambda i: (i, 0)),
        out_shape=jax.ShapeDtypeStruct((n, _NC), jnp.float32),
    )(acca, accb, xla, xra, xlb, xrb, Aa, Ab, K, S, b5)


def _gather_body(xl_hbm, xr_hbm, src_hbm, dst_hbm, outl, outr, idx_v, *rest):
    bufs = rest[:_NBUF]
    gsems = rest[_NBUF:2 * _NBUF]
    wsems = rest[2 * _NBUF:3 * _NBUF]
    wid = lax.axis_index("c") * _NSUB + lax.axis_index("s")
    base_row = wid * _EW
    for table, ind, out in ((xl_hbm, src_hbm, outl), (xr_hbm, dst_hbm, outr)):
        pltpu.sync_copy(ind.at[wid], idx_v)

        def fire_g(j, b):
            pltpu.async_copy(table.at[idx_v.at[j]], bufs[b], gsems[b])

        def drain_g(j, b):
            pltpu.make_async_copy(
                table.at[idx_v.at[j]], bufs[b], gsems[b]).wait()

        def fire_w(j, b):
            pltpu.async_copy(
                bufs[b], out.at[pl.ds(base_row + j * _C, _C)], wsems[b])

        def drain_w(j, b):
            pltpu.make_async_copy(
                bufs[b], out.at[pl.ds(base_row + j * _C, _C)], wsems[b]).wait()

        for j in range(3):
            fire_g(j, j % _NBUF)

        def outer(o, _):
            j0 = o * _UNROLL
            for u in range(_UNROLL):
                j = j0 + u
                b = u % _NBUF
                bn = (u + 3) % _NBUF

                @pl.when(jnp.logical_and(j >= 2, j + 3 < _CPW))
                def _(j=j, bn=bn):
                    drain_w(j - 2, bn)

                @pl.when(j + 3 < _CPW)
                def _(j=j, bn=bn):
                    fire_g(j + 3, bn)

                drain_g(j, b)
                fire_w(j, b)
            return 0

        lax.fori_loop(0, _CPW // _UNROLL, outer, 0)
        for t in range(_NBUF):
            j = _CPW - _NBUF + t
            drain_w(j, j % _NBUF)


def _gather_sc(xl, xr, src3, dst3):
    w = xl.shape[1]
    mesh = plsc.VectorSubcoreMesh(core_axis_name="c", subcore_axis_name="s")
    return pl.kernel(
        _gather_body,
        out_type=[jax.ShapeDtypeStruct((_E, w), jnp.float32)] * 2,
        mesh=mesh,
        compiler_params=pltpu.CompilerParams(use_tc_tiling_on_sc=False),
        scratch_types=(
            [pltpu.VMEM((_CPW, _C), jnp.int32)]
            + [pltpu.VMEM((_C, w), jnp.float32) for _ in range(_NBUF)]
            + [pltpu.SemaphoreType.DMA for _ in range(2 * _NBUF)]),
    )(xl, xr, src3, dst3)


def _scatter_body(cc, cpw, unroll, vals_hbm, dst_hbm, z_hbm, out_hbm,
                  idx_v, *rest):
    vbufs = rest[:_NBUF]
    lsems = rest[_NBUF:2 * _NBUF]
    ssems = rest[2 * _NBUF:3 * _NBUF]
    cid = lax.axis_index("c")
    sid = lax.axis_index("s")
    wid = cid * _NSUB + sid
    base_row = wid * _EW
    nch = _N // cc

    pltpu.sync_copy(z_hbm, vbufs[0])

    def zc(i, _):
        ch = sid + i * _NSUB

        @pl.when(ch < nch)
        def _():
            pltpu.sync_copy(vbufs[0], acc_sp.at[pl.ds(ch * cc, cc)])
        return 0

    acc_sp = rest[3 * _NBUF]
    lax.fori_loop(0, (nch + _NSUB - 1) // _NSUB, zc, 0)
    plsc.subcore_barrier()

    pltpu.sync_copy(dst_hbm.at[wid], idx_v)

    def fire_l(j, b):
        pltpu.async_copy(
            vals_hbm.at[pl.ds(base_row + j * cc, cc)], vbufs[b], lsems[b])

    def drain_l(j, b):
        pltpu.make_async_copy(
            vals_hbm.at[pl.ds(base_row + j * cc, cc)], vbufs[b],
            lsems[b]).wait()

    def fire_s(j, b):
        pltpu.async_copy(vbufs[b], acc_sp.at[idx_v.at[j]], ssems[b],
                         add=True)

    def drain_s(j, b):
        pltpu.make_async_copy(
            vbufs[b], acc_sp.at[idx_v.at[j]], ssems[b]).wait()

    for j in range(3):
        fire_l(j, j % _NBUF)

    def outer(o, _):
        j0 = o * unroll
        for u in range(unroll):
            j = j0 + u
            b = u % _NBUF
            bn = (u + 3) % _NBUF

            @pl.when(jnp.logical_and(j >= 2, j + 3 < cpw))
            def _(j=j, bn=bn):
                drain_s(j - 2, bn)

            @pl.when(j + 3 < cpw)
            def _(j=j, bn=bn):
                fire_l(j + 3, bn)

            drain_l(j, b)
            fire_s(j, b)
        return 0

    lax.fori_loop(0, cpw // unroll, outer, 0)
    for t in range(_NBUF):
        j = cpw - _NBUF + t
        drain_s(j, j % _NBUF)
    plsc.subcore_barrier()

    def dc(i, _):
        chn = sid + i * _NSUB

        @pl.when(chn < nch)
        def _():
            pltpu.sync_copy(acc_sp.at[pl.ds(chn * cc, cc)], vbufs[0])
            pltpu.sync_copy(
                vbufs[0], out_hbm.at[pl.ds(cid * _N + chn * cc, cc)])
        return 0

    lax.fori_loop(0, (nch + _NSUB - 1) // _NSUB, dc, 0)


def _scatter_sc(vals, dst3s, zrows):
    f = vals.shape[1]
    cc = 16 if f > 100 else 40
    cpw = _EW // cc
    unroll = 5 if cpw % 10 else 10
    mesh = plsc.VectorSubcoreMesh(core_axis_name="c", subcore_axis_name="s")
    out = pl.kernel(
        functools.partial(_scatter_body, cc, cpw, unroll),
        out_type=jax.ShapeDtypeStruct((_NCORE * _N, f), jnp.float32),
        mesh=mesh,
        compiler_params=pltpu.CompilerParams(use_tc_tiling_on_sc=False),
        scratch_types=(
            [pltpu.VMEM((cpw, cc), jnp.int32)]
            + [pltpu.VMEM((cc, f), jnp.float32) for _ in range(_NBUF)]
            + [pltpu.SemaphoreType.DMA for _ in range(2 * _NBUF)]
            + [pltpu.VMEM_SHARED((_N, f), jnp.float32)]),
    )(vals, dst3s, zrows)
    return out.reshape(_NCORE, _N, f)


def _emath_body(pad, xls_ref, xrs_ref, a_ref, k_ref, vals_ref):
    xls = xls_ref[...]
    exs = jnp.exp(_dot(_leaky(xls + xrs_ref[...]), a_ref[...]))
    parts = [_dot(exs, k_ref[...]) * xls, exs]
    if pad:
        parts.append(jnp.zeros((xls.shape[0], pad), jnp.float32))
    vals_ref[...] = jnp.concatenate(parts, axis=1)


def _emath(xls, xrs, A, K, pad):
    w = xls.shape[1]
    h = A.shape[1] + 0
    blk = 4000
    return pl.pallas_call(
        functools.partial(_emath_body, pad),
        grid=(_E // blk,),
        in_specs=[
            pl.BlockSpec((blk, w), lambda i: (i, 0)),
            pl.BlockSpec((blk, w), lambda i: (i, 0)),
            pl.BlockSpec((w, h), lambda i: (0, 0)),
            pl.BlockSpec((h, w), lambda i: (0, 0)),
        ],
        out_specs=pl.BlockSpec((blk, w + h + pad), lambda i: (i, 0)),
        out_shape=jax.ShapeDtypeStruct((_E, w + h + pad), jnp.float32),
    )(xls, xrs, A, K)


def _edge_sc(xl, xr, A, K, src3, dst3, dst3s, zrows, pad=0):
    xls, xrs = _gather_sc(xl, xr, src3, dst3)
    vals = _emath(xls, xrs, A, K, pad)
    return _scatter_sc(vals, dst3s, zrows)


def _att_mats(att):
    h, ch = att.shape
    A = (att[:, :, None] * jnp.eye(h, dtype=jnp.float32)[:, None, :]).reshape(
        h * ch, h)
    K = jnp.kron(jnp.eye(h, dtype=jnp.float32),
                 jnp.ones((1, ch), jnp.float32))
    return A, K


def kernel(x, edge_index, Wl1, Wr1, att1, b1, Wl2, Wr2, att2, b2, Wl3, Wr3,
           att3, b3, Wl4, Wr4, att4, b4, Wl5, Wr5, att5, b5, R1, rb1, R2,
           rb2, R3, rb3, R4, rb4):
    src3 = edge_index[0].reshape(_NW, _CPW, _C)
    dst3 = edge_index[1].reshape(_NW, _CPW, _C)
    dst3s16 = edge_index[1].reshape(_NW, _EW // 16, 16)
    z72 = jnp.zeros((40, 64 + _H), jnp.float32)
    z168 = jnp.zeros((16, 168), jnp.float32)

    A1, K1 = _att_mats(att1)
    A2, K2 = _att_mats(att2)
    A3, K3 = _att_mats(att3)
    A4, K4 = _att_mats(att4)
    A5a, K5 = _att_mats(att5[: _H // 2])
    A5b, _ = _att_mats(att5[_H // 2:])
    S5 = jnp.tile(jnp.eye(_NC, dtype=jnp.float32), (_H // 2, 1))

    xl, xr = _lin(x, Wl1, Wr1)
    acc = _edge_sc(xl, xr, A1, K1, src3, dst3, dst3, z72)
    h = _mid(acc, xl, xr, x, A1, K1, b1.reshape(1, -1), R1, rb1.reshape(1, -1))

    for (Wl, Wr, A, K, b, R, rb) in (
            (Wl2, Wr2, A2, K2, b2, R2, rb2),
            (Wl3, Wr3, A3, K3, b3, R3, rb3),
            (Wl4, Wr4, A4, K4, b4, R4, rb4)):
        xl, xr = _lin(h, Wl, Wr)
        acc = _edge_sc(xl, xr, A, K, src3, dst3, dst3, z72)
        h = _mid(acc, xl, xr, h, A, K, b.reshape(1, -1), R, rb.reshape(1, -1))

    hw = _H // 2 * _NC
    xla, xra = _lin(h, Wl5[:, :hw], Wr5[:, :hw])
    xlb, xrb = _lin(h, Wl5[:, hw:], Wr5[:, hw:])
    acca = _edge_sc(xla, xra, A5a, K5, src3, dst3, dst3s16, z168, pad=4)
    accb = _edge_sc(xlb, xrb, A5b, K5, src3, dst3, dst3s16, z168, pad=4)
    return _final(acca, accb, xla, xra, xlb, xrb, A5a, A5b, K5, S5,
                  b5.reshape(1, -1))


# speedup vs baseline: 40.1788x; 1.1922x over previous
import functools

import jax
import jax.numpy as jnp
from jax import lax
from jax.experimental import pallas as pl
from jax.experimental.pallas import tpu as pltpu
from jax.experimental.pallas import tpu_sc as plsc

_N = 10000
_E = 320000
_H = 8
_HID = 8
_NC = 40
_BLK = 2000

_NCORE = 2
_NSUB = 16
_NW = _NCORE * _NSUB
_EW = _E // _NW
_C = 40
_CPW = _EW // _C
_NBUF = 5
_UNROLL = 10
_DEN = 16


def _leaky(v):
    return jnp.where(v >= 0, v, 0.2 * v)


def _dot(a, b):
    return jnp.dot(a, b, preferred_element_type=jnp.float32)


def _lin_body(h_ref, wl_ref, wr_ref, xl_ref, xr_ref):
    h = h_ref[...]
    xl_ref[...] = _dot(h, wl_ref[...])
    xr_ref[...] = _dot(h, wr_ref[...])


def _lin(h, Wl, Wr):
    n, din = h.shape
    w = Wl.shape[1]
    return pl.pallas_call(
        _lin_body,
        grid=(n // _BLK,),
        in_specs=[
            pl.BlockSpec((_BLK, din), lambda i: (i, 0)),
            pl.BlockSpec((din, w), lambda i: (0, 0)),
            pl.BlockSpec((din, w), lambda i: (0, 0)),
        ],
        out_specs=[
            pl.BlockSpec((_BLK, w), lambda i: (i, 0)),
            pl.BlockSpec((_BLK, w), lambda i: (i, 0)),
        ],
        out_shape=[jax.ShapeDtypeStruct((n, w), jnp.float32)] * 2,
    )(h, Wl, Wr)


def _mid_body(acc_ref, xl_ref, xr_ref, hprev_ref, a_ref, k_ref, b_ref,
              r_ref, rb_ref, h_ref):
    xl = xl_ref[...]
    xr = xr_ref[...]
    w = xl.shape[1]
    exs = jnp.exp(_dot(_leaky(xl + xr), a_ref[...]))
    acc = acc_ref[0] + acc_ref[1]
    num = acc[:, :w] + _dot(exs, k_ref[...]) * xl
    den = acc[:, w:w + _H] + exs
    denx = _dot(den, k_ref[...]) + 1e-16
    gat = num / denx + b_ref[...]
    h_ref[...] = _leaky(gat + _dot(hprev_ref[...], r_ref[...]) + rb_ref[...])


def _mid(acc, xl, xr, hprev, A, K, b, R, rb):
    n, w = xl.shape
    din = hprev.shape[1]
    f = w + _DEN
    return pl.pallas_call(
        _mid_body,
        grid=(n // _BLK,),
        in_specs=[
            pl.BlockSpec((2, _BLK, f), lambda i: (0, i, 0)),
            pl.BlockSpec((_BLK, w), lambda i: (i, 0)),
            pl.BlockSpec((_BLK, w), lambda i: (i, 0)),
            pl.BlockSpec((_BLK, din), lambda i: (i, 0)),
            pl.BlockSpec((w, _H), lambda i: (0, 0)),
            pl.BlockSpec((_H, w), lambda i: (0, 0)),
            pl.BlockSpec((1, w), lambda i: (0, 0)),
            pl.BlockSpec((din, w), lambda i: (0, 0)),
            pl.BlockSpec((1, w), lambda i: (0, 0)),
        ],
        out_specs=pl.BlockSpec((_BLK, w), lambda i: (i, 0)),
        out_shape=jax.ShapeDtypeStruct((n, w), jnp.float32),
    )(acc, xl, xr, hprev, A, K, b, R, rb)


def _final_body(*refs):
    accs = refs[0:4]
    xls = refs[4:8]
    xrs = refs[8:12]
    aas = refs[12:16]
    k_ref, s_ref, b_ref, out_ref = refs[16:]
    w = xls[0].shape[1]

    tot = None
    for acc_ref, xl_ref, xr_ref, a_ref in zip(accs, xls, xrs, aas):
        xl = xl_ref[...]
        exs = jnp.exp(_dot(_leaky(xl + xr_ref[...]), a_ref[...]))
        acc = acc_ref[0] + acc_ref[1]
        num = acc[:, :w] + _dot(exs, k_ref[...]) * xl
        den = acc[:, w:w + 2] + exs
        gat = num / (_dot(den, k_ref[...]) + 1e-16)
        part = _dot(gat, s_ref[...])
        tot = part if tot is None else tot + part
    out = tot * 0.125 + b_ref[...]
    mx = jnp.max(out, axis=1, keepdims=True)
    lse = jnp.log(jnp.sum(jnp.exp(out - mx), axis=1, keepdims=True)) + mx
    out_ref[...] = out - lse


def _final(accs, xls, xrs, As, K, S, b5):
    n, w = xls[0].shape
    f = w + _DEN
    acc_spec = pl.BlockSpec((2, _BLK, f), lambda i: (0, i, 0))
    row_spec = pl.BlockSpec((_BLK, w), lambda i: (i, 0))
    a_spec = pl.BlockSpec((w, 2), lambda i: (0, 0))
    return pl.pallas_call(
        _final_body,
        grid=(n // _BLK,),
        in_specs=([acc_spec] * 4 + [row_spec] * 8 + [a_spec] * 4 + [
            pl.BlockSpec((2, w), lambda i: (0, 0)),
            pl.BlockSpec((w, _NC), lambda i: (0, 0)),
            pl.BlockSpec((1, _NC), lambda i: (0, 0)),
        ]),
        out_specs=pl.BlockSpec((_BLK, _NC), lambda i: (i, 0)),
        out_shape=jax.ShapeDtypeStruct((n, _NC), jnp.float32),
    )(*accs, *xls, *xrs, *As, K, S, b5)


_GD = lax.GatherDimensionNumbers(
    offset_dims=(), collapsed_slice_dims=(0,), start_index_map=(0,))


def _take(v, idx):
    return lax.gather(v, idx[:, None], _GD, (1,),
                      mode=lax.GatherScatterMode.PROMISE_IN_BOUNDS)


def _ec_ch8(gl, gr, att_v, vals, e):
    iota = lax.iota(jnp.int32, 16)
    dsel = (iota & 1) * 8
    dv = jnp.zeros((16,), jnp.float32)
    for k in range(4):
        sl = pl.ds(k * 16, 16)
        glv = gl[e, sl]
        x = glv + gr[e, sl]
        p = _leaky(x) * att_v[sl]
        for sh in (4, 2, 1):
            p = p + _take(p, iota ^ sh)
        ex = jnp.exp(p)
        vals[e, sl] = glv * ex
        dv = dv + jnp.where((iota >> 1) == k, _take(ex, dsel), 0.0)
    vals[e, pl.ds(64, _DEN)] = dv


def _ec_ch40(gl, gr, att_v, vals, e):
    iota = lax.iota(jnp.int32, 16)
    ps = []
    gls = []
    for k in range(5):
        sl = pl.ds(k * 16, 16)
        glv = gl[e, sl]
        gls.append(glv)
        x = glv + gr[e, sl]
        ps.append(_leaky(x) * att_v[sl])

    def allred(v):
        for sh in (8, 4, 2, 1):
            v = v + _take(v, iota ^ sh)
        return v

    b2 = ps[2]
    for sh in (4, 2, 1):
        b2 = b2 + _take(b2, iota ^ sh)
    e0 = allred(ps[0]) + allred(ps[1]) + _take(b2, iota & 7)
    e1 = _take(b2, iota | 8) + allred(ps[3]) + allred(ps[4])
    ex0 = jnp.exp(e0)
    ex1 = jnp.exp(e1)
    exm = jnp.where(iota < 8, ex0, ex1)
    for k, fac in enumerate((ex0, ex0, exm, ex1, ex1)):
        vals[e, pl.ds(k * 16, 16)] = gls[k] * fac
    dv = jnp.where(iota == 0, ex0, jnp.where(iota == 1, ex1, 0.0))
    vals[e, pl.ds(80, _DEN)] = dv


def _edge_body(ch, xl_hbm, xr_hbm, src_hbm, dst_hbm, att_hbm, z_hbm, out_hbm,
               att_v, *rest):
    gls = rest[0:_NBUF]
    grs = rest[_NBUF:2 * _NBUF]
    vls = rest[2 * _NBUF:3 * _NBUF]
    ixs = rest[3 * _NBUF:4 * _NBUF]
    ixd = rest[4 * _NBUF:5 * _NBUF]
    isems = rest[5 * _NBUF:6 * _NBUF]
    gsems = rest[6 * _NBUF:7 * _NBUF]
    ssems = rest[7 * _NBUF:8 * _NBUF]
    acc_sp = rest[8 * _NBUF]
    compute = _ec_ch8 if ch == 8 else _ec_ch40

    cid = lax.axis_index("c")
    sid = lax.axis_index("s")
    wid = cid * _NSUB + sid

    pltpu.sync_copy(att_hbm, att_v)

    pltpu.sync_copy(z_hbm, vls[0])

    def zc(i, _):
        chn = sid + i * _NSUB

        @pl.when(chn < _N // _C)
        def _():
            pltpu.sync_copy(vls[0], acc_sp.at[pl.ds(chn * _C, _C)])
        return 0

    lax.fori_loop(0, (_N // _C + _NSUB - 1) // _NSUB, zc, 0)
    plsc.subcore_barrier()

    def fire_i(j, b):
        pltpu.async_copy(src_hbm.at[wid, j], ixs[b], isems[b])
        pltpu.async_copy(dst_hbm.at[wid, j], ixd[b], isems[b])

    def drain_i(j, b):
        pltpu.make_async_copy(src_hbm.at[wid, j], ixs[b], isems[b]).wait()
        pltpu.make_async_copy(dst_hbm.at[wid, j], ixd[b], isems[b]).wait()

    def fire_g(j, b):
        pltpu.async_copy(xl_hbm.at[ixs[b]], gls[b], gsems[b])
        pltpu.async_copy(xr_hbm.at[ixd[b]], grs[b], gsems[b])

    def drain_g(j, b):
        pltpu.make_async_copy(xl_hbm.at[ixs[b]], gls[b], gsems[b]).wait()
        pltpu.make_async_copy(xr_hbm.at[ixd[b]], grs[b], gsems[b]).wait()

    def fire_s(j, b):
        pltpu.async_copy(vls[b], acc_sp.at[ixd[b]], ssems[b], add=True)

    def drain_s(j, b):
        pltpu.make_async_copy(vls[b], acc_sp.at[ixd[b]], ssems[b]).wait()

    for j in range(3):
        fire_i(j, j % _NBUF)
    for j in range(2):
        drain_i(j, j % _NBUF)
        fire_g(j, j % _NBUF)

    def outer(o, _):
        j0 = o * _UNROLL
        for u in range(_UNROLL):
            j = j0 + u
            b = u % _NBUF

            @pl.when(j >= 2)
            def _(j=j, b=b):
                drain_s(j - 2, (b + 3) % _NBUF)

            @pl.when(j + 3 < _CPW)
            def _(j=j, b=b):
                fire_i(j + 3, (b + 3) % _NBUF)

            @pl.when(j + 2 < _CPW)
            def _(j=j, b=b):
                drain_i(j + 2, (b + 2) % _NBUF)
                fire_g(j + 2, (b + 2) % _NBUF)

            drain_g(j, b)

            def ec(e, _, b=b):
                compute(gls[b], grs[b], att_v, vls[b], e)
                return 0

            lax.fori_loop(0, _C, ec, 0)
            fire_s(j, b)
        return 0

    lax.fori_loop(0, _CPW // _UNROLL, outer, 0)
    for j in range(_CPW - 2, _CPW):
        drain_s(j, j % _NBUF)
    plsc.subcore_barrier()

    def dc(i, _):
        chn = sid + i * _NSUB

        @pl.when(chn < _N // _C)
        def _():
            pltpu.sync_copy(acc_sp.at[pl.ds(chn * _C, _C)], vls[0])
            pltpu.sync_copy(
                vls[0], out_hbm.at[pl.ds(cid * _N + chn * _C, _C)])
        return 0

    lax.fori_loop(0, (_N // _C + _NSUB - 1) // _NSUB, dc, 0)


def _edge_sc(xl, xr, attv, src3, dst3, zrows):
    w = xl.shape[1]
    ch = 8 if w == 64 else 40
    f = w + _DEN
    mesh = plsc.VectorSubcoreMesh(core_axis_name="c", subcore_axis_name="s")
    out = pl.kernel(
        functools.partial(_edge_body, ch),
        out_type=jax.ShapeDtypeStruct((_NCORE * _N, f), jnp.float32),
        mesh=mesh,
        compiler_params=pltpu.CompilerParams(use_tc_tiling_on_sc=False),
        scratch_types=(
            [pltpu.VMEM((w,), jnp.float32)]
            + [pltpu.VMEM((_C, w), jnp.float32) for _ in range(2 * _NBUF)]
            + [pltpu.VMEM((_C, f), jnp.float32) for _ in range(_NBUF)]
            + [pltpu.VMEM((_C,), jnp.int32) for _ in range(2 * _NBUF)]
            + [pltpu.SemaphoreType.DMA for _ in range(3 * _NBUF)]
            + [pltpu.VMEM_SHARED((_N, f), jnp.float32)]),
    )(xl, xr, src3, dst3, attv, zrows)
    return out.reshape(_NCORE, _N, f)


def _att_mats(att):
    h, ch = att.shape
    A = (att[:, :, None] * jnp.eye(h, dtype=jnp.float32)[:, None, :]).reshape(
        h * ch, h)
    K = jnp.kron(jnp.eye(h, dtype=jnp.float32),
                 jnp.ones((1, ch), jnp.float32))
    return A, K


def kernel(x, edge_index, Wl1, Wr1, att1, b1, Wl2, Wr2, att2, b2, Wl3, Wr3,
           att3, b3, Wl4, Wr4, att4, b4, Wl5, Wr5, att5, b5, R1, rb1, R2,
           rb2, R3, rb3, R4, rb4):
    src3 = edge_index[0].reshape(_NW, _CPW, _C)
    dst3 = edge_index[1].reshape(_NW, _CPW, _C)
    z80 = jnp.zeros((_C, 64 + _DEN), jnp.float32)
    z96 = jnp.zeros((_C, 80 + _DEN), jnp.float32)

    A1, K1 = _att_mats(att1)
    S5 = jnp.tile(jnp.eye(_NC, dtype=jnp.float32), (2, 1))

    xl, xr = _lin(x, Wl1, Wr1)
    acc = _edge_sc(xl, xr, att1.reshape(-1), src3, dst3, z80)
    h = _mid(acc, xl, xr, x, A1, K1, b1.reshape(1, -1), R1, rb1.reshape(1, -1))

    for (Wl, Wr, att, b, R, rb) in (
            (Wl2, Wr2, att2, b2, R2, rb2),
            (Wl3, Wr3, att3, b3, R3, rb3),
            (Wl4, Wr4, att4, b4, R4, rb4)):
        A, K = _att_mats(att)
        xl, xr = _lin(h, Wl, Wr)
        acc = _edge_sc(xl, xr, att.reshape(-1), src3, dst3, z80)
        h = _mid(acc, xl, xr, h, A, K, b.reshape(1, -1), R, rb.reshape(1, -1))

    accs, xls, xrs, As = [], [], [], []
    K5 = None
    for k in range(4):
        wsl = slice(k * 80, (k + 1) * 80)
        xlk, xrk = _lin(h, Wl5[:, wsl], Wr5[:, wsl])
        attk = att5[2 * k:2 * k + 2]
        A5k, K5 = _att_mats(attk)
        acck = _edge_sc(xlk, xrk, attk.reshape(-1), src3, dst3, z96)
        accs.append(acck)
        xls.append(xlk)
        xrs.append(xrk)
        As.append(A5k)
    return _final(accs, xls, xrs, As, K5, S5, b5.reshape(1, -1))
